# baseline (speedup 1.0000x reference)
import functools

import jax
import jax.numpy as jnp
from jax import lax
from jax.experimental import pallas as pl
from jax.experimental.pallas import tpu as pltpu
from jax.experimental.pallas import tpu_sc as plsc

NC = 2
NS = 16
NW = NC * NS
K = 128
LANES = 16


def _sc_aggregate(table, src_r, dst_r, acc_rows, with_counts):
    G = 2
    _, d = table.shape
    _, C, _ = src_r.shape
    CH = -(-C // 2)
    rpt = acc_rows // NS
    nblk = rpt // K

    out_type = [jax.ShapeDtypeStruct((NC, acc_rows, d), jnp.float32)]
    if with_counts:
        out_type.append(jax.ShapeDtypeStruct((NC, acc_rows), jnp.float32))

    scratch = (
        [pltpu.VMEM((CH, K), jnp.int32),
         pltpu.VMEM((CH, K), jnp.int32)]
        + [pltpu.VMEM((K, d), jnp.float32) for _ in range(G)]
        + [pltpu.VMEM((K,), jnp.float32),
           pltpu.VMEM((K,), jnp.float32),
           pltpu.VMEM_SHARED((acc_rows, d), jnp.float32),
           pltpu.VMEM_SHARED((acc_rows,), jnp.float32)]
        + [pltpu.SemaphoreType.DMA for _ in range(2 * G + 1)]
    )
    mesh = plsc.VectorSubcoreMesh(core_axis_name="c", subcore_axis_name="s")

    def body(table_hbm, src_hbm, dst_hbm, *refs):
        if with_counts:
            sum_out, cnt_out = refs[0], refs[1]
            refs = refs[2:]
        else:
            sum_out = refs[0]
            refs = refs[1:]
        src_v, dst_v = refs[0], refs[1]
        rows = refs[2:2 + G]
        ones_v, zeros_v, acc, acc_cnt = refs[2 + G:6 + G]
        sems = refs[6 + G:]
        c = lax.axis_index("c")
        s = lax.axis_index("s")
        w = s * NC + c

        zero16 = jnp.zeros((LANES,), jnp.float32)
        one16 = jnp.ones((LANES,), jnp.float32)

        def zrow(i, carry):
            r = i // (d // LANES)
            cc = (i % (d // LANES)) * LANES
            rows[0][r, pl.ds(cc, LANES)] = zero16
            return carry

        lax.fori_loop(0, K * (d // LANES), zrow, 0)
        for i in range(K // LANES):
            ones_v[pl.ds(i * LANES, LANES)] = one16
            zeros_v[pl.ds(i * LANES, LANES)] = zero16

        r0 = pl.multiple_of(s * rpt, K)
        for b in range(nblk):
            pltpu.sync_copy(rows[0], acc.at[pl.ds(r0 + b * K, K)])
        if with_counts:
            for b in range(nblk):
                pltpu.sync_copy(zeros_v, acc_cnt.at[pl.ds(r0 + b * K, K)])
        plsc.subcore_barrier()

        def group(i, carry):
            hs = [pltpu.async_copy(table_hbm.at[src_v.at[i * G + g]], rows[g],
                                   sems[g]) for g in range(G)]
            ss = []
            for g in range(G):
                hs[g].wait()
                ss.append(pltpu.async_copy(
                    rows[g], acc.at[dst_v.at[i * G + g]], sems[G + g],
                    add=True))
                if with_counts:
                    ss.append(pltpu.async_copy(
                        ones_v, acc_cnt.at[dst_v.at[i * G + g]], sems[2 * G],
                        add=True))
            for h in ss:
                h.wait()
            return carry

        for p, span in enumerate([CH, C - CH]):
            pltpu.sync_copy(src_hbm.at[w, pl.ds(p * CH, span)],
                            src_v.at[pl.ds(0, span)])
            pltpu.sync_copy(dst_hbm.at[w, pl.ds(p * CH, span)],
                            dst_v.at[pl.ds(0, span)])
            lax.fori_loop(0, span // G, group, 0)
            for j in range(span - span % G, span):
                pltpu.async_copy(table_hbm.at[src_v.at[j]], rows[0],
                                 sems[0]).wait()
                pltpu.sync_copy(rows[0], acc.at[dst_v.at[j]], add=True)
                if with_counts:
                    pltpu.sync_copy(ones_v, acc_cnt.at[dst_v.at[j]], add=True)
        plsc.subcore_barrier()

        for b in range(nblk):
            sl = pl.ds(r0 + b * K, K)
            pltpu.sync_copy(acc.at[sl], sum_out.at[c, sl])
        if with_counts:
            for b in range(nblk):
                sl = pl.ds(r0 + b * K, K)
                pltpu.sync_copy(acc_cnt.at[sl], cnt_out.at[c, sl])

    fn = pl.kernel(
        body,
        mesh=mesh,
        out_type=tuple(out_type) if with_counts else out_type[0],
        scratch_types=scratch,
    )
    return fn(table, src_r, dst_r)


def _matT(a, w):
    return lax.dot_general(a, w, (((1,), (1,)), ((), ())),
                           preferred_element_type=jnp.float32)


def _tc_project(x, Wp, bp, bn):
    n, d = x.shape

    def body(x_ref, w_ref, b_ref, o_ref):
        o_ref[...] = jnp.maximum(_matT(x_ref[...], w_ref[...]) + b_ref[...], 0.0)

    return pl.pallas_call(
        body,
        grid=(n // bn,),
        in_specs=[pl.BlockSpec((bn, d), lambda i: (i, 0)),
                  pl.BlockSpec((d, d), lambda i: (0, 0)),
                  pl.BlockSpec((1, d), lambda i: (0, 0))],
        out_specs=pl.BlockSpec((bn, d), lambda i: (i, 0)),
        out_shape=jax.ShapeDtypeStruct((n, d), jnp.float32),
    )(x, Wp, bp.reshape(1, d))


def _tc_layer1(sums, cnts, x, Wl1, bl1, Wr1, ln_g, ln_b, bn):
    n, d = x.shape

    def body(s_ref, c_ref, x_ref, wl_ref, bl_ref, wr_ref, g_ref, b_ref, o_ref):
        sarr = s_ref[...]
        carr = c_ref[...]
        cnt = jnp.maximum(carr[0] + carr[1], 1.0)
        aggr = (sarr[0] + sarr[1]) / cnt[:, None]
        out = _matT(aggr, wl_ref[...]) + bl_ref[...] + _matT(x_ref[...], wr_ref[...])
        nrm = jnp.sqrt(jnp.sum(out * out, axis=1, keepdims=True))
        out = out / jnp.maximum(nrm, 1e-12)
        out = jnp.maximum(out, 0.0)
        mu = jnp.mean(out, axis=1, keepdims=True)
        var = jnp.mean((out - mu) ** 2, axis=1, keepdims=True)
        o_ref[...] = (out - mu) * lax.rsqrt(var + 1e-5) * g_ref[...] + b_ref[...]

    return pl.pallas_call(
        body,
        grid=(n // bn,),
        in_specs=[pl.BlockSpec((NC, bn, d), lambda i: (0, i, 0)),
                  pl.BlockSpec((NC, bn), lambda i: (0, i)),
                  pl.BlockSpec((bn, d), lambda i: (i, 0)),
                  pl.BlockSpec((d, d), lambda i: (0, 0)),
                  pl.BlockSpec((1, d), lambda i: (0, 0)),
                  pl.BlockSpec((d, d), lambda i: (0, 0)),
                  pl.BlockSpec((1, d), lambda i: (0, 0)),
                  pl.BlockSpec((1, d), lambda i: (0, 0))],
        out_specs=pl.BlockSpec((bn, d), lambda i: (i, 0)),
        out_shape=jax.ShapeDtypeStruct((n, d), jnp.float32),
    )(sums, cnts, x, Wl1, bl1.reshape(1, d), Wr1,
      ln_g.reshape(1, d), ln_b.reshape(1, d))


def _tc_layer2(sums, cnts, y, Wl2, bl2, Wr2, bn):
    n, d = y.shape

    def body(s_ref, c_ref, y_ref, wl_ref, bl_ref, wr_ref, o_ref):
        sarr = s_ref[...]
        carr = c_ref[...]
        cnt = jnp.maximum(carr[0] + carr[1], 1.0)
        aggr = (sarr[0] + sarr[1]) / cnt[:, None]
        o_ref[...] = (_matT(aggr, wl_ref[...]) + bl_ref[...]
                      + _matT(y_ref[...], wr_ref[...]))

    return pl.pallas_call(
        body,
        grid=(n // bn,),
        in_specs=[pl.BlockSpec((NC, bn, d), lambda i: (0, i, 0)),
                  pl.BlockSpec((NC, bn), lambda i: (0, i)),
                  pl.BlockSpec((bn, d), lambda i: (i, 0)),
                  pl.BlockSpec((d, d), lambda i: (0, 0)),
                  pl.BlockSpec((1, d), lambda i: (0, 0)),
                  pl.BlockSpec((d, d), lambda i: (0, 0))],
        out_specs=pl.BlockSpec((bn, d), lambda i: (i, 0)),
        out_shape=jax.ShapeDtypeStruct((n, d), jnp.float32),
    )(sums, cnts, y, Wl2, bl2.reshape(1, d), Wr2)


def kernel(x, edge_index, Wp, bp, Wl1, bl1, Wr1, ln_g, ln_b, Wl2, bl2, Wr2):
    n, d = x.shape
    e = edge_index.shape[1]

    C = (-(-e // (NW * K)) + 1) // 2 * 2
    e_pad = NW * K * C
    src = jnp.pad(edge_index[0], (0, e_pad - e))
    dst = jnp.pad(edge_index[1], (0, e_pad - e), constant_values=n)
    src_r = src.reshape(NW, C, K)
    dst_r = dst.reshape(NW, C, K)

    acc_rows = -(-(n + 1) // (NS * K)) * (NS * K)
    bn = acc_rows // 10
    xp = jnp.pad(x, ((0, acc_rows - n), (0, 0)))

    h = _tc_project(xp, Wp, bp, bn)
    sums, cnts = _sc_aggregate(h, src_r, dst_r, acc_rows, True)
    out = _tc_layer1(sums, cnts, xp, Wl1, bl1, Wr1, ln_g, ln_b, bn)
    sums2 = _sc_aggregate(out, src_r, dst_r, acc_rows, False)
    out2 = _tc_layer2(sums2, cnts, out, Wl2, bl2, Wr2, bn)
    return out2[:n]


# speedup vs baseline: 1.0023x; 1.0023x over previous
import functools

import jax
import jax.numpy as jnp
from jax import lax
from jax.experimental import pallas as pl
from jax.experimental.pallas import tpu as pltpu
from jax.experimental.pallas import tpu_sc as plsc

NC = 2
NS = 16
NW = NC * NS
K = 128
LANES = 16


def _sc_aggregate(table, src_r, dst_r, acc_rows, with_counts):
    G = 2
    _, d = table.shape
    _, C, _ = src_r.shape
    CH = -(-C // 2)
    rpt = acc_rows // NS
    nblk = rpt // K

    out_type = [jax.ShapeDtypeStruct((NC, acc_rows, d), jnp.float32)]
    if with_counts:
        out_type.append(jax.ShapeDtypeStruct((NC, acc_rows), jnp.float32))

    scratch = (
        [pltpu.VMEM((CH, K), jnp.int32),
         pltpu.VMEM((CH, K), jnp.int32)]
        + [pltpu.VMEM((K, d), jnp.float32) for _ in range(G)]
        + [pltpu.VMEM((K,), jnp.float32),
           pltpu.VMEM((K,), jnp.float32),
           pltpu.VMEM_SHARED((acc_rows, d), jnp.float32),
           pltpu.VMEM_SHARED((acc_rows,), jnp.float32)]
        + [pltpu.SemaphoreType.DMA for _ in range(2 * G + 1)]
    )
    mesh = plsc.VectorSubcoreMesh(core_axis_name="c", subcore_axis_name="s")

    def body(table_hbm, src_hbm, dst_hbm, *refs):
        if with_counts:
            sum_out, cnt_out = refs[0], refs[1]
            refs = refs[2:]
        else:
            sum_out = refs[0]
            refs = refs[1:]
        src_v, dst_v = refs[0], refs[1]
        rows = refs[2:2 + G]
        ones_v, zeros_v, acc, acc_cnt = refs[2 + G:6 + G]
        sems = refs[6 + G:]
        c = lax.axis_index("c")
        s = lax.axis_index("s")
        w = s * NC + c

        zero16 = jnp.zeros((LANES,), jnp.float32)
        one16 = jnp.ones((LANES,), jnp.float32)

        def zrow(i, carry):
            r = i // (d // LANES)
            cc = (i % (d // LANES)) * LANES
            rows[0][r, pl.ds(cc, LANES)] = zero16
            return carry

        lax.fori_loop(0, K * (d // LANES), zrow, 0)
        for i in range(K // LANES):
            ones_v[pl.ds(i * LANES, LANES)] = one16
            zeros_v[pl.ds(i * LANES, LANES)] = zero16

        r0 = pl.multiple_of(s * rpt, K)
        for b in range(nblk):
            pltpu.sync_copy(rows[0], acc.at[pl.ds(r0 + b * K, K)])
        if with_counts:
            for b in range(nblk):
                pltpu.sync_copy(zeros_v, acc_cnt.at[pl.ds(r0 + b * K, K)])
        plsc.subcore_barrier()

        def group(i, carry):
            hs = [pltpu.async_copy(table_hbm.at[src_v.at[i * G + g]], rows[g],
                                   sems[g]) for g in range(G)]
            ss = []
            for g in range(G):
                hs[g].wait()
                ss.append(pltpu.async_copy(
                    rows[g], acc.at[dst_v.at[i * G + g]], sems[G + g],
                    add=True))
                if with_counts:
                    ss.append(pltpu.async_copy(
                        ones_v, acc_cnt.at[dst_v.at[i * G + g]], sems[2 * G],
                        add=True))
            for h in ss:
                h.wait()
            return carry

        for p, span in enumerate([CH, C - CH]):
            pltpu.sync_copy(src_hbm.at[w, pl.ds(p * CH, span)],
                            src_v.at[pl.ds(0, span)])
            pltpu.sync_copy(dst_hbm.at[w, pl.ds(p * CH, span)],
                            dst_v.at[pl.ds(0, span)])
            lax.fori_loop(0, span // G, group, 0)
            for j in range(span - span % G, span):
                pltpu.async_copy(table_hbm.at[src_v.at[j]], rows[0],
                                 sems[0]).wait()
                pltpu.sync_copy(rows[0], acc.at[dst_v.at[j]], add=True)
                if with_counts:
                    pltpu.sync_copy(ones_v, acc_cnt.at[dst_v.at[j]], add=True)
        plsc.subcore_barrier()

        for b in range(nblk):
            sl = pl.ds(r0 + b * K, K)
            pltpu.sync_copy(acc.at[sl], sum_out.at[c, sl])
        if with_counts:
            for b in range(nblk):
                sl = pl.ds(r0 + b * K, K)
                pltpu.sync_copy(acc_cnt.at[sl], cnt_out.at[c, sl])

    fn = pl.kernel(
        body,
        mesh=mesh,
        out_type=tuple(out_type) if with_counts else out_type[0],
        scratch_types=scratch,
    )
    return fn(table, src_r, dst_r)


def _matT(a, w):
    return lax.dot_general(a, w, (((1,), (1,)), ((), ())),
                           preferred_element_type=jnp.float32)


def _tc_project(x, Wp, bp, bn):
    n, d = x.shape

    def body(x_ref, w_ref, b_ref, o_ref):
        o_ref[...] = jnp.maximum(_matT(x_ref[...], w_ref[...]) + b_ref[...], 0.0)

    return pl.pallas_call(
        body,
        grid=(n // bn,),
        in_specs=[pl.BlockSpec((bn, d), lambda i: (i, 0)),
                  pl.BlockSpec((d, d), lambda i: (0, 0)),
                  pl.BlockSpec((1, d), lambda i: (0, 0))],
        out_specs=pl.BlockSpec((bn, d), lambda i: (i, 0)),
        out_shape=jax.ShapeDtypeStruct((n, d), jnp.float32),
    )(x, Wp, bp.reshape(1, d))


def _tc_layer1(sums, cnts, x, Wl1, bl1, Wr1, ln_g, ln_b, bn):
    n, d = x.shape

    def body(s_ref, c_ref, x_ref, wl_ref, bl_ref, wr_ref, g_ref, b_ref, o_ref):
        sarr = s_ref[...]
        carr = c_ref[...]
        cnt = jnp.maximum(carr[0] + carr[1], 1.0)
        aggr = (sarr[0] + sarr[1]) / cnt[:, None]
        out = _matT(aggr, wl_ref[...]) + bl_ref[...] + _matT(x_ref[...], wr_ref[...])
        nrm = jnp.sqrt(jnp.sum(out * out, axis=1, keepdims=True))
        out = out / jnp.maximum(nrm, 1e-12)
        out = jnp.maximum(out, 0.0)
        mu = jnp.mean(out, axis=1, keepdims=True)
        var = jnp.mean((out - mu) ** 2, axis=1, keepdims=True)
        o_ref[...] = (out - mu) * lax.rsqrt(var + 1e-5) * g_ref[...] + b_ref[...]

    return pl.pallas_call(
        body,
        grid=(n // bn,),
        in_specs=[pl.BlockSpec((NC, bn, d), lambda i: (0, i, 0)),
                  pl.BlockSpec((NC, bn), lambda i: (0, i)),
                  pl.BlockSpec((bn, d), lambda i: (i, 0)),
                  pl.BlockSpec((d, d), lambda i: (0, 0)),
                  pl.BlockSpec((1, d), lambda i: (0, 0)),
                  pl.BlockSpec((d, d), lambda i: (0, 0)),
                  pl.BlockSpec((1, d), lambda i: (0, 0)),
                  pl.BlockSpec((1, d), lambda i: (0, 0))],
        out_specs=pl.BlockSpec((bn, d), lambda i: (i, 0)),
        out_shape=jax.ShapeDtypeStruct((n, d), jnp.float32),
    )(sums, cnts, x, Wl1, bl1.reshape(1, d), Wr1,
      ln_g.reshape(1, d), ln_b.reshape(1, d))


def _tc_layer2(sums, cnts, y, Wl2, bl2, Wr2, bn):
    n, d = y.shape

    def body(s_ref, c_ref, y_ref, wl_ref, bl_ref, wr_ref, o_ref):
        sarr = s_ref[...]
        carr = c_ref[...]
        cnt = jnp.maximum(carr[0] + carr[1], 1.0)
        aggr = (sarr[0] + sarr[1]) / cnt[:, None]
        o_ref[...] = (_matT(aggr, wl_ref[...]) + bl_ref[...]
                      + _matT(y_ref[...], wr_ref[...]))

    return pl.pallas_call(
        body,
        grid=(n // bn,),
        in_specs=[pl.BlockSpec((NC, bn, d), lambda i: (0, i, 0)),
                  pl.BlockSpec((NC, bn), lambda i: (0, i)),
                  pl.BlockSpec((bn, d), lambda i: (i, 0)),
                  pl.BlockSpec((d, d), lambda i: (0, 0)),
                  pl.BlockSpec((1, d), lambda i: (0, 0)),
                  pl.BlockSpec((d, d), lambda i: (0, 0))],
        out_specs=pl.BlockSpec((bn, d), lambda i: (i, 0)),
        out_shape=jax.ShapeDtypeStruct((n, d), jnp.float32),
    )(sums, cnts, y, Wl2, bl2.reshape(1, d), Wr2)


def kernel(x, edge_index, Wp, bp, Wl1, bl1, Wr1, ln_g, ln_b, Wl2, bl2, Wr2):
    n, d = x.shape
    e = edge_index.shape[1]

    C = (-(-e // (NW * K)) + 1) // 2 * 2
    e_pad = NW * K * C
    src = jnp.pad(edge_index[0], (0, e_pad - e))
    dst = jnp.concatenate(
        [edge_index[1],
         n + (jnp.arange(e_pad - e, dtype=jnp.int32) % 128)])
    src_r = src.reshape(NW, C, K)
    dst_r = dst.reshape(NW, C, K)

    acc_rows = -(-(n + 128) // (NS * K)) * (NS * K)
    bn = acc_rows // 10
    xp = jnp.pad(x, ((0, acc_rows - n), (0, 0)))

    h = _tc_project(xp, Wp, bp, bn)
    sums, cnts = _sc_aggregate(h, src_r, dst_r, acc_rows, True)
    out = _tc_layer1(sums, cnts, xp, Wl1, bl1, Wr1, ln_g, ln_b, bn)
    sums2 = _sc_aggregate(out, src_r, dst_r, acc_rows, False)
    out2 = _tc_layer2(sums2, cnts, out, Wl2, bl2, Wr2, bn)
    return out2[:n]


# speedup vs baseline: 3.0591x; 3.0520x over previous
import functools

import jax
import jax.numpy as jnp
from jax import lax
from jax.experimental import pallas as pl
from jax.experimental.pallas import tpu as pltpu
from jax.experimental.pallas import tpu_sc as plsc

NC = 2
NS = 16
NW = NC * NS
K = 128
LANES = 16


def _sc_aggregate(table, src_r, dst_r, acc_rows, with_counts):
    G = 2
    _, d = table.shape
    _, C, _ = src_r.shape
    CH = -(-C // 2)
    rpt = acc_rows // NS
    nblk = rpt // K

    out_type = [jax.ShapeDtypeStruct((NC, acc_rows, d), jnp.float32)]
    if with_counts:
        out_type.append(jax.ShapeDtypeStruct((NC, acc_rows), jnp.float32))

    scratch = (
        [pltpu.VMEM((CH, K), jnp.int32),
         pltpu.VMEM((CH, K), jnp.int32)]
        + [pltpu.VMEM((K, d), jnp.float32) for _ in range(G)]
        + [pltpu.VMEM((K,), jnp.float32),
           pltpu.VMEM((K,), jnp.float32),
           pltpu.VMEM_SHARED((acc_rows, d), jnp.float32),
           pltpu.VMEM_SHARED((acc_rows,), jnp.float32)]
        + [pltpu.SemaphoreType.DMA for _ in range(2 * G + 1)]
    )
    mesh = plsc.VectorSubcoreMesh(core_axis_name="c", subcore_axis_name="s")

    def body(table_hbm, src_hbm, dst_hbm, *refs):
        if with_counts:
            sum_out, cnt_out = refs[0], refs[1]
            refs = refs[2:]
        else:
            sum_out = refs[0]
            refs = refs[1:]
        src_v, dst_v = refs[0], refs[1]
        rows = refs[2:2 + G]
        ones_v, zeros_v, acc, acc_cnt = refs[2 + G:6 + G]
        sems = refs[6 + G:]
        c = lax.axis_index("c")
        s = lax.axis_index("s")
        w = s * NC + c

        zero16 = jnp.zeros((LANES,), jnp.float32)
        one16 = jnp.ones((LANES,), jnp.float32)

        def zrow(i, carry):
            r = i // (d // LANES)
            cc = (i % (d // LANES)) * LANES
            rows[0][r, pl.ds(cc, LANES)] = zero16
            return carry

        lax.fori_loop(0, K * (d // LANES), zrow, 0)
        for i in range(K // LANES):
            ones_v[pl.ds(i * LANES, LANES)] = one16
            zeros_v[pl.ds(i * LANES, LANES)] = zero16

        r0 = pl.multiple_of(s * rpt, K)
        for b in range(nblk):
            pltpu.sync_copy(rows[0], acc.at[pl.ds(r0 + b * K, K)])
        if with_counts:
            for b in range(nblk):
                pltpu.sync_copy(zeros_v, acc_cnt.at[pl.ds(r0 + b * K, K)])
        plsc.subcore_barrier()

        def group(i, carry):
            hs = [pltpu.async_copy(table_hbm.at[src_v.at[i * G + g]], rows[g],
                                   sems[g]) for g in range(G)]
            ss = []
            for g in range(G):
                hs[g].wait()
                ss.append(pltpu.async_copy(
                    rows[g], acc.at[dst_v.at[i * G + g]], sems[G + g],
                    add=True))
                if with_counts:
                    ss.append(pltpu.async_copy(
                        ones_v, acc_cnt.at[dst_v.at[i * G + g]], sems[2 * G],
                        add=True))
            for h in ss:
                h.wait()
            return carry

        for p, span in enumerate([CH, C - CH]):
            pltpu.sync_copy(src_hbm.at[w, pl.ds(p * CH, span)],
                            src_v.at[pl.ds(0, span)])
            pltpu.sync_copy(dst_hbm.at[w, pl.ds(p * CH, span)],
                            dst_v.at[pl.ds(0, span)])
            lax.fori_loop(0, span // G, group, 0)
            for j in range(span - span % G, span):
                pltpu.async_copy(table_hbm.at[src_v.at[j]], rows[0],
                                 sems[0]).wait()
                pltpu.sync_copy(rows[0], acc.at[dst_v.at[j]], add=True)
                if with_counts:
                    pltpu.sync_copy(ones_v, acc_cnt.at[dst_v.at[j]], add=True)
        plsc.subcore_barrier()

        for b in range(nblk):
            sl = pl.ds(r0 + b * K, K)
            pltpu.sync_copy(acc.at[sl], sum_out.at[c, sl])
        if with_counts:
            for b in range(nblk):
                sl = pl.ds(r0 + b * K, K)
                pltpu.sync_copy(acc_cnt.at[sl], cnt_out.at[c, sl])

    fn = pl.kernel(
        body,
        mesh=mesh,
        out_type=tuple(out_type) if with_counts else out_type[0],
        scratch_types=scratch,
    )
    return fn(table, src_r, dst_r)


def _matT(a, w):
    return lax.dot_general(a, w, (((1,), (1,)), ((), ())),
                           preferred_element_type=jnp.float32)


def _tc_project(x, Wp, bp, bn):
    n, d = x.shape

    def body(x_ref, w_ref, b_ref, o_ref):
        o_ref[...] = jnp.maximum(_matT(x_ref[...], w_ref[...]) + b_ref[...], 0.0)

    return pl.pallas_call(
        body,
        grid=(n // bn,),
        in_specs=[pl.BlockSpec((bn, d), lambda i: (i, 0)),
                  pl.BlockSpec((d, d), lambda i: (0, 0)),
                  pl.BlockSpec((1, d), lambda i: (0, 0))],
        out_specs=pl.BlockSpec((bn, d), lambda i: (i, 0)),
        out_shape=jax.ShapeDtypeStruct((n, d), jnp.float32),
    )(x, Wp, bp.reshape(1, d))


def _tc_layer1(sums, cnts, x, Wl1, bl1, Wr1, ln_g, ln_b, bn):
    n, d = x.shape

    def body(s_ref, c_ref, x_ref, wl_ref, bl_ref, wr_ref, g_ref, b_ref, o_ref):
        sarr = s_ref[...]
        carr = c_ref[...]
        cnt = jnp.maximum(carr[0] + carr[1], 1.0)
        aggr = (sarr[0] + sarr[1]) / cnt[:, None]
        out = _matT(aggr, wl_ref[...]) + bl_ref[...] + _matT(x_ref[...], wr_ref[...])
        nrm = jnp.sqrt(jnp.sum(out * out, axis=1, keepdims=True))
        out = out / jnp.maximum(nrm, 1e-12)
        out = jnp.maximum(out, 0.0)
        mu = jnp.mean(out, axis=1, keepdims=True)
        var = jnp.mean((out - mu) ** 2, axis=1, keepdims=True)
        o_ref[...] = (out - mu) * lax.rsqrt(var + 1e-5) * g_ref[...] + b_ref[...]

    return pl.pallas_call(
        body,
        grid=(n // bn,),
        in_specs=[pl.BlockSpec((NC, bn, d), lambda i: (0, i, 0)),
                  pl.BlockSpec((NC, bn), lambda i: (0, i)),
                  pl.BlockSpec((bn, d), lambda i: (i, 0)),
                  pl.BlockSpec((d, d), lambda i: (0, 0)),
                  pl.BlockSpec((1, d), lambda i: (0, 0)),
                  pl.BlockSpec((d, d), lambda i: (0, 0)),
                  pl.BlockSpec((1, d), lambda i: (0, 0)),
                  pl.BlockSpec((1, d), lambda i: (0, 0))],
        out_specs=pl.BlockSpec((bn, d), lambda i: (i, 0)),
        out_shape=jax.ShapeDtypeStruct((n, d), jnp.float32),
    )(sums, cnts, x, Wl1, bl1.reshape(1, d), Wr1,
      ln_g.reshape(1, d), ln_b.reshape(1, d))


def _tc_layer2(sums, cnts, y, Wl2, bl2, Wr2, bn):
    n, d = y.shape

    def body(s_ref, c_ref, y_ref, wl_ref, bl_ref, wr_ref, o_ref):
        sarr = s_ref[...]
        carr = c_ref[...]
        cnt = jnp.maximum(carr[0] + carr[1], 1.0)
        aggr = (sarr[0] + sarr[1]) / cnt[:, None]
        o_ref[...] = (_matT(aggr, wl_ref[...]) + bl_ref[...]
                      + _matT(y_ref[...], wr_ref[...]))

    return pl.pallas_call(
        body,
        grid=(n // bn,),
        in_specs=[pl.BlockSpec((NC, bn, d), lambda i: (0, i, 0)),
                  pl.BlockSpec((NC, bn), lambda i: (0, i)),
                  pl.BlockSpec((bn, d), lambda i: (i, 0)),
                  pl.BlockSpec((d, d), lambda i: (0, 0)),
                  pl.BlockSpec((1, d), lambda i: (0, 0)),
                  pl.BlockSpec((d, d), lambda i: (0, 0))],
        out_specs=pl.BlockSpec((bn, d), lambda i: (i, 0)),
        out_shape=jax.ShapeDtypeStruct((n, d), jnp.float32),
    )(sums, cnts, y, Wl2, bl2.reshape(1, d), Wr2)


def kernel(x, edge_index, Wp, bp, Wl1, bl1, Wr1, ln_g, ln_b, Wl2, bl2, Wr2):
    n, d = x.shape
    e = edge_index.shape[1]

    C = (-(-e // (NW * K)) + 1) // 2 * 2
    e_pad = NW * K * C
    pad_i = jnp.arange(e_pad - e, dtype=jnp.int32)
    src = jnp.concatenate([edge_index[0], pad_i % n])
    dst = jnp.concatenate([edge_index[1], n + pad_i % 128])
    src_r = src.reshape(NW, C, K)
    dst_r = dst.reshape(NW, C, K)

    acc_rows = -(-(n + 128) // (NS * K)) * (NS * K)
    bn = acc_rows // 10
    xp = jnp.pad(x, ((0, acc_rows - n), (0, 0)))

    h = _tc_project(xp, Wp, bp, bn)
    sums, cnts = _sc_aggregate(h, src_r, dst_r, acc_rows, True)
    out = _tc_layer1(sums, cnts, xp, Wl1, bl1, Wr1, ln_g, ln_b, bn)
    sums2 = _sc_aggregate(out, src_r, dst_r, acc_rows, False)
    out2 = _tc_layer2(sums2, cnts, out, Wl2, bl2, Wr2, bn)
    return out2[:n]


# speedup vs baseline: 3.1278x; 1.0225x over previous
import functools

import jax
import jax.numpy as jnp
from jax import lax
from jax.experimental import pallas as pl
from jax.experimental.pallas import tpu as pltpu
from jax.experimental.pallas import tpu_sc as plsc

NC = 2
NS = 16
NW = NC * NS
K = 128
LANES = 16


def _sc_aggregate(table, src_r, dst_r, acc_rows, with_counts):
    G = 2
    _, d = table.shape
    _, C, _ = src_r.shape
    CH = -(-C // 2)
    rpt = acc_rows // NS
    nblk = rpt // K

    out_type = [jax.ShapeDtypeStruct((NC, acc_rows, d), jnp.float32)]
    if with_counts:
        out_type.append(jax.ShapeDtypeStruct((NC, acc_rows), jnp.float32))

    scratch = (
        [pltpu.VMEM((CH, K), jnp.int32),
         pltpu.VMEM((CH, K), jnp.int32)]
        + [pltpu.VMEM((K, d), jnp.float32) for _ in range(G)]
        + [pltpu.VMEM((K,), jnp.float32),
           pltpu.VMEM((K,), jnp.float32),
           pltpu.VMEM_SHARED((acc_rows, d), jnp.float32),
           pltpu.VMEM_SHARED((acc_rows,), jnp.float32)]
        + [pltpu.SemaphoreType.DMA for _ in range(2 * G + 1)]
    )
    mesh = plsc.VectorSubcoreMesh(core_axis_name="c", subcore_axis_name="s")

    def body(table_hbm, src_hbm, dst_hbm, *refs):
        if with_counts:
            sum_out, cnt_out = refs[0], refs[1]
            refs = refs[2:]
        else:
            sum_out = refs[0]
            refs = refs[1:]
        src_v, dst_v = refs[0], refs[1]
        rows = refs[2:2 + G]
        ones_v, zeros_v, acc, acc_cnt = refs[2 + G:6 + G]
        sems = refs[6 + G:]
        c = lax.axis_index("c")
        s = lax.axis_index("s")
        w = s * NC + c

        zero16 = jnp.zeros((LANES,), jnp.float32)
        one16 = jnp.ones((LANES,), jnp.float32)

        def zrow(i, carry):
            r = i // (d // LANES)
            cc = (i % (d // LANES)) * LANES
            rows[0][r, pl.ds(cc, LANES)] = zero16
            return carry

        lax.fori_loop(0, K * (d // LANES), zrow, 0)
        for i in range(K // LANES):
            ones_v[pl.ds(i * LANES, LANES)] = one16
            zeros_v[pl.ds(i * LANES, LANES)] = zero16

        r0 = pl.multiple_of(s * rpt, K)
        for b in range(nblk):
            pltpu.sync_copy(rows[0], acc.at[pl.ds(r0 + b * K, K)])
        if with_counts:
            for b in range(nblk):
                pltpu.sync_copy(zeros_v, acc_cnt.at[pl.ds(r0 + b * K, K)])
        plsc.subcore_barrier()

        def group(i, carry):
            hs = [pltpu.async_copy(table_hbm.at[src_v.at[i * G + g]], rows[g],
                                   sems[g]) for g in range(G)]
            ss = []
            for g in range(G):
                hs[g].wait()
                ss.append(pltpu.async_copy(
                    rows[g], acc.at[dst_v.at[i * G + g]], sems[G + g],
                    add=True))
                if with_counts:
                    ss.append(pltpu.async_copy(
                        ones_v, acc_cnt.at[dst_v.at[i * G + g]], sems[2 * G],
                        add=True))
            for h in ss:
                h.wait()
            return carry

        for p, span in enumerate([CH, C - CH]):
            pltpu.sync_copy(src_hbm.at[w, pl.ds(p * CH, span)],
                            src_v.at[pl.ds(0, span)])
            pltpu.sync_copy(dst_hbm.at[w, pl.ds(p * CH, span)],
                            dst_v.at[pl.ds(0, span)])
            lax.fori_loop(0, span // G, group, 0)
            for j in range(span - span % G, span):
                pltpu.async_copy(table_hbm.at[src_v.at[j]], rows[0],
                                 sems[0]).wait()
                pltpu.sync_copy(rows[0], acc.at[dst_v.at[j]], add=True)
                if with_counts:
                    pltpu.sync_copy(ones_v, acc_cnt.at[dst_v.at[j]], add=True)
        plsc.subcore_barrier()

        for b in range(nblk):
            sl = pl.ds(r0 + b * K, K)
            pltpu.sync_copy(acc.at[sl], sum_out.at[c, sl])
        if with_counts:
            for b in range(nblk):
                sl = pl.ds(r0 + b * K, K)
                pltpu.sync_copy(acc_cnt.at[sl], cnt_out.at[c, sl])

    fn = pl.kernel(
        body,
        mesh=mesh,
        out_type=tuple(out_type) if with_counts else out_type[0],
        scratch_types=scratch,
    )
    return fn(table, src_r, dst_r)


def _matT(a, w):
    return lax.dot_general(a, w, (((1,), (1,)), ((), ())),
                           preferred_element_type=jnp.float32)


def _tc_project(x, Wp, bp, bn):
    n, d = x.shape

    def body(x_ref, w_ref, b_ref, o_ref):
        o_ref[...] = jnp.maximum(_matT(x_ref[...], w_ref[...]) + b_ref[...], 0.0)

    return pl.pallas_call(
        body,
        grid=(n // bn,),
        in_specs=[pl.BlockSpec((bn, d), lambda i: (i, 0)),
                  pl.BlockSpec((d, d), lambda i: (0, 0)),
                  pl.BlockSpec((1, d), lambda i: (0, 0))],
        out_specs=pl.BlockSpec((bn, d), lambda i: (i, 0)),
        out_shape=jax.ShapeDtypeStruct((n, d), jnp.float32),
    )(x, Wp, bp.reshape(1, d))


def _tc_layer1(sums, cnts, x, Wl1, bl1, Wr1, ln_g, ln_b, bn):
    n, d = x.shape

    def body(s_ref, c_ref, x_ref, wl_ref, bl_ref, wr_ref, g_ref, b_ref, o_ref):
        sarr = s_ref[...]
        carr = c_ref[...]
        cnt = jnp.maximum(carr[0] + carr[1], 1.0)
        aggr = (sarr[0] + sarr[1]) / cnt[:, None]
        out = _matT(aggr, wl_ref[...]) + bl_ref[...] + _matT(x_ref[...], wr_ref[...])
        nrm = jnp.sqrt(jnp.sum(out * out, axis=1, keepdims=True))
        out = out / jnp.maximum(nrm, 1e-12)
        out = jnp.maximum(out, 0.0)
        mu = jnp.mean(out, axis=1, keepdims=True)
        var = jnp.mean((out - mu) ** 2, axis=1, keepdims=True)
        o_ref[...] = (out - mu) * lax.rsqrt(var + 1e-5) * g_ref[...] + b_ref[...]

    return pl.pallas_call(
        body,
        grid=(n // bn,),
        in_specs=[pl.BlockSpec((NC, bn, d), lambda i: (0, i, 0)),
                  pl.BlockSpec((NC, bn), lambda i: (0, i)),
                  pl.BlockSpec((bn, d), lambda i: (i, 0)),
                  pl.BlockSpec((d, d), lambda i: (0, 0)),
                  pl.BlockSpec((1, d), lambda i: (0, 0)),
                  pl.BlockSpec((d, d), lambda i: (0, 0)),
                  pl.BlockSpec((1, d), lambda i: (0, 0)),
                  pl.BlockSpec((1, d), lambda i: (0, 0))],
        out_specs=pl.BlockSpec((bn, d), lambda i: (i, 0)),
        out_shape=jax.ShapeDtypeStruct((n, d), jnp.float32),
    )(sums, cnts, x, Wl1, bl1.reshape(1, d), Wr1,
      ln_g.reshape(1, d), ln_b.reshape(1, d))


def _tc_layer2(sums, cnts, y, Wl2, bl2, Wr2, bn):
    n, d = y.shape

    def body(s_ref, c_ref, y_ref, wl_ref, bl_ref, wr_ref, o_ref):
        sarr = s_ref[...]
        carr = c_ref[...]
        cnt = jnp.maximum(carr[0] + carr[1], 1.0)
        aggr = (sarr[0] + sarr[1]) / cnt[:, None]
        o_ref[...] = (_matT(aggr, wl_ref[...]) + bl_ref[...]
                      + _matT(y_ref[...], wr_ref[...]))

    return pl.pallas_call(
        body,
        grid=(n // bn,),
        in_specs=[pl.BlockSpec((NC, bn, d), lambda i: (0, i, 0)),
                  pl.BlockSpec((NC, bn), lambda i: (0, i)),
                  pl.BlockSpec((bn, d), lambda i: (i, 0)),
                  pl.BlockSpec((d, d), lambda i: (0, 0)),
                  pl.BlockSpec((1, d), lambda i: (0, 0)),
                  pl.BlockSpec((d, d), lambda i: (0, 0))],
        out_specs=pl.BlockSpec((bn, d), lambda i: (i, 0)),
        out_shape=jax.ShapeDtypeStruct((n, d), jnp.float32),
    )(sums, cnts, y, Wl2, bl2.reshape(1, d), Wr2)


def kernel(x, edge_index, Wp, bp, Wl1, bl1, Wr1, ln_g, ln_b, Wl2, bl2, Wr2):
    n, d = x.shape
    e = edge_index.shape[1]

    C = (-(-e // (NW * K)) + 1) // 2 * 2
    e_pad = NW * K * C
    pad_i = jnp.arange(e_pad - e, dtype=jnp.int32)
    src = jnp.concatenate([edge_index[0], pad_i % n])
    dst = jnp.concatenate([edge_index[1], n + pad_i % 128])
    src_r = src.reshape(NW, C, K)
    dst_r = dst.reshape(NW, C, K)

    acc_rows = -(-(n + 128) // (NS * K)) * (NS * K)
    bn = acc_rows // 5
    xp = jnp.pad(x, ((0, acc_rows - n), (0, 0)))

    h = _tc_project(xp, Wp, bp, bn)
    sums, cnts = _sc_aggregate(h, src_r, dst_r, acc_rows, True)
    out = _tc_layer1(sums, cnts, xp, Wl1, bl1, Wr1, ln_g, ln_b, bn)
    sums2 = _sc_aggregate(out, src_r, dst_r, acc_rows, False)
    out2 = _tc_layer2(sums2, cnts, out, Wl2, bl2, Wr2, bn)
    return out2[:n]


# speedup vs baseline: 3.1553x; 1.0088x over previous
import functools

import jax
import jax.numpy as jnp
from jax import lax
from jax.experimental import pallas as pl
from jax.experimental.pallas import tpu as pltpu
from jax.experimental.pallas import tpu_sc as plsc

NC = 2
NS = 16
NW = NC * NS
K = 128
LANES = 16


def _sc_aggregate(table, ei, acc_rows, with_counts):
    G = 2
    _, d = table.shape
    C = ei.shape[2]
    CH = -(-C // 2)
    rpt = acc_rows // NS
    nblk = rpt // K

    out_type = [jax.ShapeDtypeStruct((NC, acc_rows, d), jnp.float32)]
    if with_counts:
        out_type.append(jax.ShapeDtypeStruct((NC, acc_rows), jnp.float32))

    scratch = (
        [pltpu.VMEM((CH, K), jnp.int32),
         pltpu.VMEM((CH, K), jnp.int32)]
        + [pltpu.VMEM((K, d), jnp.float32) for _ in range(G)]
        + [pltpu.VMEM((K,), jnp.float32),
           pltpu.VMEM((K,), jnp.float32),
           pltpu.VMEM_SHARED((acc_rows, d), jnp.float32),
           pltpu.VMEM_SHARED((acc_rows,), jnp.float32)]
        + [pltpu.SemaphoreType.DMA for _ in range(2 * G + 1)]
    )
    mesh = plsc.VectorSubcoreMesh(core_axis_name="c", subcore_axis_name="s")

    def body(table_hbm, ei_hbm, *refs):
        if with_counts:
            sum_out, cnt_out = refs[0], refs[1]
            refs = refs[2:]
        else:
            sum_out = refs[0]
            refs = refs[1:]
        src_v, dst_v = refs[0], refs[1]
        rows = refs[2:2 + G]
        ones_v, zeros_v, acc, acc_cnt = refs[2 + G:6 + G]
        sems = refs[6 + G:]
        c = lax.axis_index("c")
        s = lax.axis_index("s")
        w = s * NC + c

        zero16 = jnp.zeros((LANES,), jnp.float32)
        one16 = jnp.ones((LANES,), jnp.float32)

        def zrow(i, carry):
            r = i // (d // LANES)
            cc = (i % (d // LANES)) * LANES
            rows[0][r, pl.ds(cc, LANES)] = zero16
            return carry

        lax.fori_loop(0, K * (d // LANES), zrow, 0)
        for i in range(K // LANES):
            ones_v[pl.ds(i * LANES, LANES)] = one16
            zeros_v[pl.ds(i * LANES, LANES)] = zero16

        r0 = pl.multiple_of(s * rpt, K)
        for b in range(nblk):
            pltpu.sync_copy(rows[0], acc.at[pl.ds(r0 + b * K, K)])
        if with_counts:
            for b in range(nblk):
                pltpu.sync_copy(zeros_v, acc_cnt.at[pl.ds(r0 + b * K, K)])
        plsc.subcore_barrier()

        def group(i, carry):
            hs = [pltpu.async_copy(table_hbm.at[src_v.at[i * G + g]], rows[g],
                                   sems[g]) for g in range(G)]
            ss = []
            for g in range(G):
                hs[g].wait()
                ss.append(pltpu.async_copy(
                    rows[g], acc.at[dst_v.at[i * G + g]], sems[G + g],
                    add=True))
                if with_counts:
                    ss.append(pltpu.async_copy(
                        ones_v, acc_cnt.at[dst_v.at[i * G + g]], sems[2 * G],
                        add=True))
            for h in ss:
                h.wait()
            return carry

        for p, span in enumerate([CH, C - CH]):
            pltpu.sync_copy(ei_hbm.at[0, w, pl.ds(p * CH, span)],
                            src_v.at[pl.ds(0, span)])
            pltpu.sync_copy(ei_hbm.at[1, w, pl.ds(p * CH, span)],
                            dst_v.at[pl.ds(0, span)])
            lax.fori_loop(0, span // G, group, 0)
            for j in range(span - span % G, span):
                pltpu.async_copy(table_hbm.at[src_v.at[j]], rows[0],
                                 sems[0]).wait()
                pltpu.sync_copy(rows[0], acc.at[dst_v.at[j]], add=True)
                if with_counts:
                    pltpu.sync_copy(ones_v, acc_cnt.at[dst_v.at[j]], add=True)
        plsc.subcore_barrier()

        for b in range(nblk):
            sl = pl.ds(r0 + b * K, K)
            pltpu.sync_copy(acc.at[sl], sum_out.at[c, sl])
        if with_counts:
            for b in range(nblk):
                sl = pl.ds(r0 + b * K, K)
                pltpu.sync_copy(acc_cnt.at[sl], cnt_out.at[c, sl])

    fn = pl.kernel(
        body,
        mesh=mesh,
        out_type=tuple(out_type) if with_counts else out_type[0],
        scratch_types=scratch,
    )
    return fn(table, ei)


def _matT(a, w):
    return lax.dot_general(a, w, (((1,), (1,)), ((), ())),
                           preferred_element_type=jnp.float32)


def _tc_project(x, Wp, bp, bn):
    n, d = x.shape

    def body(x_ref, w_ref, b_ref, o_ref):
        o_ref[...] = jnp.maximum(_matT(x_ref[...], w_ref[...]) + b_ref[...], 0.0)

    return pl.pallas_call(
        body,
        grid=(n // bn,),
        in_specs=[pl.BlockSpec((bn, d), lambda i: (i, 0)),
                  pl.BlockSpec((d, d), lambda i: (0, 0)),
                  pl.BlockSpec((1, d), lambda i: (0, 0))],
        out_specs=pl.BlockSpec((bn, d), lambda i: (i, 0)),
        out_shape=jax.ShapeDtypeStruct((n, d), jnp.float32),
    )(x, Wp, bp.reshape(1, d))


def _tc_layer1(sums, cnts, x, Wl1, bl1, Wr1, ln_g, ln_b, bn):
    n, d = x.shape

    def body(s_ref, c_ref, x_ref, wl_ref, bl_ref, wr_ref, g_ref, b_ref, o_ref):
        sarr = s_ref[...]
        carr = c_ref[...]
        cnt = jnp.maximum(carr[0] + carr[1], 1.0)
        aggr = (sarr[0] + sarr[1]) / cnt[:, None]
        out = _matT(aggr, wl_ref[...]) + bl_ref[...] + _matT(x_ref[...], wr_ref[...])
        nrm = jnp.sqrt(jnp.sum(out * out, axis=1, keepdims=True))
        out = out / jnp.maximum(nrm, 1e-12)
        out = jnp.maximum(out, 0.0)
        mu = jnp.mean(out, axis=1, keepdims=True)
        var = jnp.mean((out - mu) ** 2, axis=1, keepdims=True)
        o_ref[...] = (out - mu) * lax.rsqrt(var + 1e-5) * g_ref[...] + b_ref[...]

    return pl.pallas_call(
        body,
        grid=(n // bn,),
        in_specs=[pl.BlockSpec((NC, bn, d), lambda i: (0, i, 0)),
                  pl.BlockSpec((NC, bn), lambda i: (0, i)),
                  pl.BlockSpec((bn, d), lambda i: (i, 0)),
                  pl.BlockSpec((d, d), lambda i: (0, 0)),
                  pl.BlockSpec((1, d), lambda i: (0, 0)),
                  pl.BlockSpec((d, d), lambda i: (0, 0)),
                  pl.BlockSpec((1, d), lambda i: (0, 0)),
                  pl.BlockSpec((1, d), lambda i: (0, 0))],
        out_specs=pl.BlockSpec((bn, d), lambda i: (i, 0)),
        out_shape=jax.ShapeDtypeStruct((n, d), jnp.float32),
    )(sums, cnts, x, Wl1, bl1.reshape(1, d), Wr1,
      ln_g.reshape(1, d), ln_b.reshape(1, d))


def _tc_layer2(sums, cnts, y, Wl2, bl2, Wr2, bn):
    n, d = y.shape

    def body(s_ref, c_ref, y_ref, wl_ref, bl_ref, wr_ref, o_ref):
        sarr = s_ref[...]
        carr = c_ref[...]
        cnt = jnp.maximum(carr[0] + carr[1], 1.0)
        aggr = (sarr[0] + sarr[1]) / cnt[:, None]
        o_ref[...] = (_matT(aggr, wl_ref[...]) + bl_ref[...]
                      + _matT(y_ref[...], wr_ref[...]))

    return pl.pallas_call(
        body,
        grid=(n // bn,),
        in_specs=[pl.BlockSpec((NC, bn, d), lambda i: (0, i, 0)),
                  pl.BlockSpec((NC, bn), lambda i: (0, i)),
                  pl.BlockSpec((bn, d), lambda i: (i, 0)),
                  pl.BlockSpec((d, d), lambda i: (0, 0)),
                  pl.BlockSpec((1, d), lambda i: (0, 0)),
                  pl.BlockSpec((d, d), lambda i: (0, 0))],
        out_specs=pl.BlockSpec((bn, d), lambda i: (i, 0)),
        out_shape=jax.ShapeDtypeStruct((n, d), jnp.float32),
    )(sums, cnts, y, Wl2, bl2.reshape(1, d), Wr2)


def kernel(x, edge_index, Wp, bp, Wl1, bl1, Wr1, ln_g, ln_b, Wl2, bl2, Wr2):
    n, d = x.shape
    e = edge_index.shape[1]

    C = (-(-e // (NW * K)) + 1) // 2 * 2
    e_pad = NW * K * C
    pad_i = jnp.arange(e_pad - e, dtype=jnp.int32)
    pad_blk = jnp.stack([pad_i % n, n + pad_i % 128])
    ei = jnp.concatenate([edge_index, pad_blk], axis=1).reshape(2, NW, C, K)

    acc_rows = -(-(n + 128) // (NS * K)) * (NS * K)
    bn = acc_rows // 5
    xp = jnp.pad(x, ((0, acc_rows - n), (0, 0)))

    h = _tc_project(xp, Wp, bp, bn)
    sums, cnts = _sc_aggregate(h, ei, acc_rows, True)
    out = _tc_layer1(sums, cnts, xp, Wl1, bl1, Wr1, ln_g, ln_b, bn)
    sums2 = _sc_aggregate(out, ei, acc_rows, False)
    out2 = _tc_layer2(sums2, cnts, out, Wl2, bl2, Wr2, bn)
    return out2[:n]


# speedup vs baseline: 3.1841x; 1.0091x over previous
import functools

import jax
import jax.numpy as jnp
from jax import lax
from jax.experimental import pallas as pl
from jax.experimental.pallas import tpu as pltpu
from jax.experimental.pallas import tpu_sc as plsc

NC = 2
NS = 16
NW = NC * NS
K = 128
LANES = 16


def _sc_aggregate(table, ei, acc_rows, with_counts):
    G = 2
    _, d = table.shape
    C = ei.shape[2]
    CH = -(-C // 2)
    rpt = acc_rows // NS
    nblk = rpt // K

    out_type = [jax.ShapeDtypeStruct((NC, acc_rows, d), jnp.float32)]
    if with_counts:
        out_type.append(jax.ShapeDtypeStruct((NC, acc_rows), jnp.float32))

    scratch = (
        [pltpu.VMEM((CH, K), jnp.int32),
         pltpu.VMEM((CH, K), jnp.int32)]
        + [pltpu.VMEM((K, d), jnp.float32) for _ in range(G)]
        + [pltpu.VMEM((K,), jnp.float32),
           pltpu.VMEM((K,), jnp.float32),
           pltpu.VMEM_SHARED((acc_rows, d), jnp.float32),
           pltpu.VMEM_SHARED((acc_rows,), jnp.float32)]
        + [pltpu.SemaphoreType.DMA for _ in range(2 * G + 1)]
    )
    mesh = plsc.VectorSubcoreMesh(core_axis_name="c", subcore_axis_name="s")

    def body(table_hbm, ei_hbm, *refs):
        if with_counts:
            sum_out, cnt_out = refs[0], refs[1]
            refs = refs[2:]
        else:
            sum_out = refs[0]
            refs = refs[1:]
        src_v, dst_v = refs[0], refs[1]
        rows = refs[2:2 + G]
        ones_v, zeros_v, acc, acc_cnt = refs[2 + G:6 + G]
        sems = refs[6 + G:]
        c = lax.axis_index("c")
        s = lax.axis_index("s")
        w = s * NC + c

        zero16 = jnp.zeros((LANES,), jnp.float32)
        one16 = jnp.ones((LANES,), jnp.float32)

        def zrow(i, carry):
            r = i // (d // LANES)
            cc = (i % (d // LANES)) * LANES
            rows[0][r, pl.ds(cc, LANES)] = zero16
            return carry

        lax.fori_loop(0, K * (d // LANES), zrow, 0)
        for i in range(K // LANES):
            ones_v[pl.ds(i * LANES, LANES)] = one16
            zeros_v[pl.ds(i * LANES, LANES)] = zero16

        r0 = pl.multiple_of(s * rpt, K)
        for b in range(nblk):
            pltpu.sync_copy(rows[0], acc.at[pl.ds(r0 + b * K, K)])
        if with_counts:
            for b in range(nblk):
                pltpu.sync_copy(zeros_v, acc_cnt.at[pl.ds(r0 + b * K, K)])
        plsc.subcore_barrier()

        def group(i, carry):
            hs = [pltpu.async_copy(table_hbm.at[src_v.at[i * G + g]], rows[g],
                                   sems[g]) for g in range(G)]
            ss = []
            for g in range(G):
                hs[g].wait()
                ss.append(pltpu.async_copy(
                    rows[g], acc.at[dst_v.at[i * G + g]], sems[G + g],
                    add=True))
                if with_counts:
                    ss.append(pltpu.async_copy(
                        ones_v, acc_cnt.at[dst_v.at[i * G + g]], sems[2 * G],
                        add=True))
            for h in ss:
                h.wait()
            return carry

        for p, span in enumerate([CH, C - CH]):
            pltpu.sync_copy(ei_hbm.at[0, w, pl.ds(p * CH, span)],
                            src_v.at[pl.ds(0, span)])
            pltpu.sync_copy(ei_hbm.at[1, w, pl.ds(p * CH, span)],
                            dst_v.at[pl.ds(0, span)])
            lax.fori_loop(0, span // G, group, 0)
            for j in range(span - span % G, span):
                pltpu.async_copy(table_hbm.at[src_v.at[j]], rows[0],
                                 sems[0]).wait()
                pltpu.sync_copy(rows[0], acc.at[dst_v.at[j]], add=True)
                if with_counts:
                    pltpu.sync_copy(ones_v, acc_cnt.at[dst_v.at[j]], add=True)
        plsc.subcore_barrier()

        for b in range(nblk):
            sl = pl.ds(r0 + b * K, K)
            pltpu.sync_copy(acc.at[sl], sum_out.at[c, sl])
        if with_counts:
            for b in range(nblk):
                sl = pl.ds(r0 + b * K, K)
                pltpu.sync_copy(acc_cnt.at[sl], cnt_out.at[c, sl])

    fn = pl.kernel(
        body,
        mesh=mesh,
        out_type=tuple(out_type) if with_counts else out_type[0],
        scratch_types=scratch,
    )
    return fn(table, ei)


def _matT(a, w):
    return lax.dot_general(a, w, (((1,), (1,)), ((), ())),
                           preferred_element_type=jnp.float32)


def _tc_project(x, Wp, bp, bn):
    n, d = x.shape

    def body(x_ref, w_ref, b_ref, o_ref):
        o_ref[...] = jnp.maximum(_matT(x_ref[...], w_ref[...]) + b_ref[...], 0.0)

    return pl.pallas_call(
        body,
        grid=(n // bn,),
        in_specs=[pl.BlockSpec((bn, d), lambda i: (i, 0)),
                  pl.BlockSpec((d, d), lambda i: (0, 0)),
                  pl.BlockSpec((1, d), lambda i: (0, 0))],
        out_specs=pl.BlockSpec((bn, d), lambda i: (i, 0)),
        out_shape=jax.ShapeDtypeStruct((n, d), jnp.float32),
    )(x, Wp, bp.reshape(1, d))


def _tc_layer1(sums, cnts, x, Wl1, bl1, Wr1, ln_g, ln_b, bn):
    n, d = x.shape

    def body(s_ref, c_ref, x_ref, wl_ref, bl_ref, wr_ref, g_ref, b_ref, o_ref):
        sarr = s_ref[...]
        carr = c_ref[...]
        cnt = jnp.maximum(carr[0] + carr[1], 1.0)
        aggr = (sarr[0] + sarr[1]) / cnt[:, None]
        out = _matT(aggr, wl_ref[...]) + bl_ref[...] + _matT(x_ref[...], wr_ref[...])
        nrm = jnp.sqrt(jnp.sum(out * out, axis=1, keepdims=True))
        out = out / jnp.maximum(nrm, 1e-12)
        out = jnp.maximum(out, 0.0)
        mu = jnp.mean(out, axis=1, keepdims=True)
        var = jnp.mean((out - mu) ** 2, axis=1, keepdims=True)
        o_ref[...] = (out - mu) * lax.rsqrt(var + 1e-5) * g_ref[...] + b_ref[...]

    return pl.pallas_call(
        body,
        grid=(n // bn,),
        in_specs=[pl.BlockSpec((NC, bn, d), lambda i: (0, i, 0)),
                  pl.BlockSpec((NC, bn), lambda i: (0, i)),
                  pl.BlockSpec((bn, d), lambda i: (i, 0)),
                  pl.BlockSpec((d, d), lambda i: (0, 0)),
                  pl.BlockSpec((1, d), lambda i: (0, 0)),
                  pl.BlockSpec((d, d), lambda i: (0, 0)),
                  pl.BlockSpec((1, d), lambda i: (0, 0)),
                  pl.BlockSpec((1, d), lambda i: (0, 0))],
        out_specs=pl.BlockSpec((bn, d), lambda i: (i, 0)),
        out_shape=jax.ShapeDtypeStruct((n, d), jnp.float32),
    )(sums, cnts, x, Wl1, bl1.reshape(1, d), Wr1,
      ln_g.reshape(1, d), ln_b.reshape(1, d))


def _tc_layer2(sums, cnts, y, Wl2, bl2, Wr2, bn):
    n, d = y.shape

    def body(s_ref, c_ref, y_ref, wl_ref, bl_ref, wr_ref, o_ref):
        sarr = s_ref[...]
        carr = c_ref[...]
        cnt = jnp.maximum(carr[0] + carr[1], 1.0)
        aggr = (sarr[0] + sarr[1]) / cnt[:, None]
        o_ref[...] = (_matT(aggr, wl_ref[...]) + bl_ref[...]
                      + _matT(y_ref[...], wr_ref[...]))

    return pl.pallas_call(
        body,
        grid=(n // bn,),
        in_specs=[pl.BlockSpec((NC, bn, d), lambda i: (0, i, 0)),
                  pl.BlockSpec((NC, bn), lambda i: (0, i)),
                  pl.BlockSpec((bn, d), lambda i: (i, 0)),
                  pl.BlockSpec((d, d), lambda i: (0, 0)),
                  pl.BlockSpec((1, d), lambda i: (0, 0)),
                  pl.BlockSpec((d, d), lambda i: (0, 0))],
        out_specs=pl.BlockSpec((bn, d), lambda i: (i, 0)),
        out_shape=jax.ShapeDtypeStruct((n, d), jnp.float32),
    )(sums, cnts, y, Wl2, bl2.reshape(1, d), Wr2)


def kernel(x, edge_index, Wp, bp, Wl1, bl1, Wr1, ln_g, ln_b, Wl2, bl2, Wr2):
    n, d = x.shape
    e = edge_index.shape[1]

    C = (-(-e // (NW * K)) + 1) // 2 * 2
    e_pad = NW * K * C
    pad_i = jnp.arange(e_pad - e, dtype=jnp.int32)
    pad_blk = jnp.stack([pad_i % n, n + pad_i % 128])
    ei = jnp.concatenate([edge_index, pad_blk], axis=1).reshape(2, NW, C, K)

    acc_rows = -(-(n + 128) // (NS * K)) * (NS * K)
    bn = acc_rows // 4
    xp = jnp.pad(x, ((0, acc_rows - n), (0, 0)))

    h = _tc_project(xp, Wp, bp, bn)
    sums, cnts = _sc_aggregate(h, ei, acc_rows, True)
    out = _tc_layer1(sums, cnts, xp, Wl1, bl1, Wr1, ln_g, ln_b, bn)
    sums2 = _sc_aggregate(out, ei, acc_rows, False)
    out2 = _tc_layer2(sums2, cnts, out, Wl2, bl2, Wr2, bn)
    return out2[:n]
